# baseline (speedup 1.0000x reference)
import dataclasses
import functools

import jax
import jax.numpy as jnp
from jax import lax
from jax.experimental import pallas as pl
from jax.experimental.pallas import tpu as pltpu
from jax.experimental.pallas import tpu_sc as plsc

HIGHEST = lax.Precision.HIGHEST

_SC_CP = pltpu.CompilerParams()
if "needs_layout_passes" in pltpu.CompilerParams.__dataclass_fields__:
    _SC_CP = dataclasses.replace(_SC_CP, needs_layout_passes=False)

NTILE = 16
NCORE = 2
CHUNK = 128
BLK = 1024


def _dot_t(a, b):
    return lax.dot_general(a, b, (((1,), (1,)), ((), ())),
                           precision=HIGHEST,
                           preferred_element_type=jnp.float32)


def _sc_mesh():
    return plsc.VectorSubcoreMesh(core_axis_name="c", subcore_axis_name="s")


def _make_deg_kernel(n_pad, per_tile):
    nrow = n_pad // 128

    @functools.partial(
        pl.kernel,
        mesh=_sc_mesh(),
        out_type=jax.ShapeDtypeStruct((NCORE, nrow, 128), jnp.float32),
        compiler_params=_SC_CP,
        scratch_types=[
            pltpu.VMEM((per_tile,), jnp.int32),
            pltpu.VMEM((nrow, 128), jnp.float32),
            pltpu.VMEM((nrow,), jnp.int32),
            pltpu.VMEM_SHARED((nrow, 128), jnp.float32),
        ],
    )
    def deg_kernel(dst_hbm, zeros_hbm, degp_hbm, idx_v, hist_v, rows_v, acc):
        c = lax.axis_index("c")
        s = lax.axis_index("s")
        pltpu.sync_copy(dst_hbm.at[c, s], idx_v)
        pltpu.sync_copy(zeros_hbm.at[pl.ds(0, nrow)], hist_v)

        @pl.when(s == 0)
        def _():
            pltpu.sync_copy(zeros_hbm.at[pl.ds(0, nrow)], acc)

        iota = lax.iota(jnp.int32, 16)

        @pl.loop(0, nrow // 16)
        def _(j):
            rows_v[pl.ds(j * 16, 16)] = iota + j * 16

        ones = jnp.ones((16,), jnp.float32)

        @pl.loop(0, per_tile // 16)
        def _(i):
            iv = idx_v[pl.ds(i * 16, 16)]
            plsc.addupdate_scatter(
                hist_v, [lax.shift_right_logical(iv, 7), iv & 127], ones)

        plsc.subcore_barrier()
        pltpu.sync_copy(hist_v, acc.at[rows_v], add=True)
        plsc.subcore_barrier()

        @pl.when(s == 0)
        def _():
            pltpu.sync_copy(acc, degp_hbm.at[c])

    return deg_kernel


def _make_scatter_kernel(n_pad, nchunk, hh, feature_split):
    rpt = n_pad // NTILE
    sup = 40
    assert nchunk % sup == 0
    nsup = nchunk // sup

    @functools.partial(
        pl.kernel,
        mesh=_sc_mesh(),
        out_type=jax.ShapeDtypeStruct((NCORE * n_pad, hh), jnp.float32),
        scratch_types=[
            pltpu.VMEM((sup, CHUNK), jnp.int32),
            pltpu.VMEM((sup, CHUNK), jnp.int32),
            pltpu.VMEM((CHUNK, hh), jnp.float32),
            pltpu.VMEM((CHUNK, hh), jnp.float32),
            pltpu.VMEM_SHARED((n_pad, hh), jnp.float32),
            pltpu.SemaphoreType.DMA,
            pltpu.SemaphoreType.DMA,
        ],
    )
    def scatter_kernel(u_hbm, src_hbm, dst_hbm, zeros_hbm, v_hbm,
                       src_v, dst_v, g0, g1, acc, gsem0, gsem1):
        c = lax.axis_index("c")
        s = lax.axis_index("s")
        if feature_split:
            pltpu.sync_copy(u_hbm.at[pl.ds(c * n_pad + s * rpt, rpt)],
                            acc.at[pl.ds(s * rpt, rpt)])
        else:
            pltpu.sync_copy(zeros_hbm, acc.at[pl.ds(s * rpt, rpt)])
        plsc.subcore_barrier()

        def wait_gather(buf, sem):
            pltpu.make_async_copy(u_hbm.at[src_v.at[0]], buf, sem).wait()

        @pl.loop(0, nsup)
        def _(sc):
            pltpu.sync_copy(src_hbm.at[c, s, pl.ds(sc * sup, sup)], src_v)
            pltpu.sync_copy(dst_hbm.at[c, s, pl.ds(sc * sup, sup)], dst_v)
            pltpu.async_copy(u_hbm.at[src_v.at[0]], g0, gsem0)

            @pl.loop(0, sup, step=2)
            def _(j):
                pltpu.async_copy(u_hbm.at[src_v.at[j + 1]], g1, gsem1)
                wait_gather(g0, gsem0)
                pltpu.sync_copy(g0, acc.at[dst_v.at[j]], add=True)

                @pl.when(j + 2 < sup)
                def _():
                    pltpu.async_copy(u_hbm.at[src_v.at[j + 2]], g0, gsem0)

                wait_gather(g1, gsem1)
                pltpu.sync_copy(g1, acc.at[dst_v.at[j + 1]], add=True)

        plsc.subcore_barrier()
        pltpu.sync_copy(acc.at[pl.ds(s * rpt, rpt)],
                        v_hbm.at[pl.ds(c * n_pad + s * rpt, rpt)])

    return scatter_kernel


def _mm1_body(x_ref, w_ref, h_ref):
    h_ref[...] = _dot_t(x_ref[...], w_ref[...])


def _dis(degp_ref):
    deg = degp_ref[:, 0:1] + degp_ref[:, 1:2] + 1.0
    return lax.rsqrt(deg)


def _scale_body(h_ref, degp_ref, u_ref, *, hh):
    u = _dis(degp_ref) * h_ref[...]
    u_ref[0] = u[:, :hh]
    u_ref[1] = u[:, hh:]


def _ln_leaky(o, g_ref, be_ref):
    mu = jnp.mean(o, axis=1, keepdims=True)
    d = o - mu
    var = jnp.mean(d * d, axis=1, keepdims=True)
    xn = d * lax.rsqrt(var + 1e-5) * g_ref[...] + be_ref[...]
    return jnp.where(xn >= 0, xn, 0.01 * xn)


def _mid_body(v_ref, degp_ref, b_ref, g_ref, be_ref, w2_ref, u_ref):
    dis = _dis(degp_ref)
    v = jnp.concatenate([v_ref[0], v_ref[1]], axis=1)
    a = _ln_leaky(dis * v + b_ref[...], g_ref, be_ref)
    u2 = dis * _dot_t(a, w2_ref[...])
    u_ref[0] = u2
    u_ref[1] = u2


def _final_body(v_ref, u_ref, degp_ref, b_ref, g_ref, be_ref, batch_ref,
                wl_ref, bl_ref, out_ref, sums_ref, cnt_ref, *, num_graphs,
                nblocks):
    i = pl.program_id(0)

    @pl.when(i == 0)
    def _():
        sums_ref[...] = jnp.zeros_like(sums_ref)
        cnt_ref[...] = jnp.zeros_like(cnt_ref)

    dis = _dis(degp_ref)
    v = v_ref[0] + v_ref[1] + u_ref[0]
    h = _ln_leaky(dis * v + b_ref[...], g_ref, be_ref)

    gid = lax.broadcasted_iota(jnp.int32, (h.shape[0], num_graphs), 1)
    onehot = (batch_ref[...] == gid).astype(jnp.float32)
    sums_ref[...] += lax.dot_general(
        onehot, h, (((0,), (0,)), ((), ())),
        precision=HIGHEST, preferred_element_type=jnp.float32)
    cnt_ref[...] += lax.dot_general(
        onehot, jnp.ones((h.shape[0], 1), jnp.float32),
        (((0,), (0,)), ((), ())),
        precision=HIGHEST, preferred_element_type=jnp.float32)

    @pl.when(i == nblocks - 1)
    def _():
        pooled = sums_ref[...] / jnp.maximum(cnt_ref[...], 1.0)
        out_ref[...] = jnp.sum(pooled * wl_ref[...], axis=1,
                               keepdims=True) + bl_ref[0, 0]


def kernel(x, edge_index, batch, W1, b1, g1, be1, W2, b2, g2, be2, Wl, bl):
    n, d = x.shape
    h1 = W1.shape[0]
    h2 = W2.shape[0]
    num_graphs = 128
    e = edge_index.shape[1]

    n_pad = ((n + BLK - 1) // BLK) * BLK
    nblocks = n_pad // BLK
    rpt = n_pad // NTILE

    grain = NCORE * NTILE * CHUNK
    e_pad = ((e + grain - 1) // grain) * grain
    per_tile_m = e_pad // NTILE
    nchunk_m = per_tile_m // CHUNK
    per_tile_d = e_pad // (NCORE * NTILE)
    nchunk_d = per_tile_d // CHUNK

    src = edge_index[0]
    dst = edge_index[1]
    sink = jnp.int32(n)
    src_p = jnp.concatenate([src, jnp.zeros((e_pad - e,), jnp.int32)])
    dst_p = jnp.concatenate([dst, jnp.full((e_pad - e,), sink, jnp.int32)])

    srcs1 = jnp.stack([src_p, src_p + n_pad]).reshape(
        NCORE, NTILE, nchunk_m, CHUNK)
    dsts1 = jnp.stack([dst_p, dst_p]).reshape(NCORE, NTILE, nchunk_m, CHUNK)
    srcs2 = (src_p.reshape(NCORE, -1)
             + jnp.array([[0], [n_pad]], jnp.int32)).reshape(
                 NCORE, NTILE, nchunk_m // 2, CHUNK)
    dsts2 = dst_p.reshape(NCORE, NTILE, nchunk_m // 2, CHUNK)
    dstd = dst_p.reshape(NCORE, NTILE, per_tile_d)

    zeros128 = jnp.zeros((rpt, 128), jnp.float32)

    x_pad = jnp.pad(x, ((0, n_pad - n), (0, 0)))
    batch_pad = jnp.pad(batch, (0, n_pad - n),
                        constant_values=-1).reshape(n_pad, 1)

    degp = _make_deg_kernel(n_pad, per_tile_d)(dstd, zeros128)
    deg_t = jnp.transpose(degp.reshape(NCORE, n_pad))

    mm1 = pl.pallas_call(
        _mm1_body,
        grid=(nblocks,),
        in_specs=[
            pl.BlockSpec((BLK, d), lambda i: (i, 0)),
            pl.BlockSpec((h1, d), lambda i: (0, 0)),
        ],
        out_specs=pl.BlockSpec((BLK, h1), lambda i: (i, 0)),
        out_shape=jax.ShapeDtypeStruct((n_pad, h1), jnp.float32),
    )
    h1_full = mm1(x_pad, W1)

    scale = pl.pallas_call(
        functools.partial(_scale_body, hh=h1 // 2),
        grid=(nblocks,),
        in_specs=[
            pl.BlockSpec((BLK, h1), lambda i: (i, 0)),
            pl.BlockSpec((BLK, NCORE), lambda i: (i, 0)),
        ],
        out_specs=pl.BlockSpec((NCORE, BLK, h1 // 2), lambda i: (0, i, 0)),
        out_shape=jax.ShapeDtypeStruct((NCORE, n_pad, h1 // 2), jnp.float32),
    )
    u1 = scale(h1_full, deg_t).reshape(NCORE * n_pad, h1 // 2)

    v1 = _make_scatter_kernel(n_pad, nchunk_m, h1 // 2, True)(
        u1, srcs1, dsts1, zeros128)
    v1 = v1.reshape(NCORE, n_pad, h1 // 2)

    mid = pl.pallas_call(
        _mid_body,
        grid=(nblocks,),
        in_specs=[
            pl.BlockSpec((NCORE, BLK, h1 // 2), lambda i: (0, i, 0)),
            pl.BlockSpec((BLK, NCORE), lambda i: (i, 0)),
            pl.BlockSpec((1, h1), lambda i: (0, 0)),
            pl.BlockSpec((1, h1), lambda i: (0, 0)),
            pl.BlockSpec((1, h1), lambda i: (0, 0)),
            pl.BlockSpec((h2, h1), lambda i: (0, 0)),
        ],
        out_specs=pl.BlockSpec((NCORE, BLK, h2), lambda i: (0, i, 0)),
        out_shape=jax.ShapeDtypeStruct((NCORE, n_pad, h2), jnp.float32),
    )
    u2d = mid(v1, deg_t, b1.reshape(1, h1), g1.reshape(1, h1),
              be1.reshape(1, h1), W2)
    u2 = u2d.reshape(NCORE * n_pad, h2)

    v2 = _make_scatter_kernel(n_pad, nchunk_m // 2, h2, False)(
        u2, srcs2, dsts2, zeros128)
    v2 = v2.reshape(NCORE, n_pad, h2)

    final = pl.pallas_call(
        functools.partial(_final_body, num_graphs=num_graphs,
                          nblocks=nblocks),
        grid=(nblocks,),
        in_specs=[
            pl.BlockSpec((NCORE, BLK, h2), lambda i: (0, i, 0)),
            pl.BlockSpec((NCORE, BLK, h2), lambda i: (0, i, 0)),
            pl.BlockSpec((BLK, NCORE), lambda i: (i, 0)),
            pl.BlockSpec((1, h2), lambda i: (0, 0)),
            pl.BlockSpec((1, h2), lambda i: (0, 0)),
            pl.BlockSpec((1, h2), lambda i: (0, 0)),
            pl.BlockSpec((BLK, 1), lambda i: (i, 0)),
            pl.BlockSpec((1, h2), lambda i: (0, 0)),
            pl.BlockSpec((1, 1), lambda i: (0, 0)),
        ],
        out_specs=pl.BlockSpec((num_graphs, 1), lambda i: (0, 0)),
        out_shape=jax.ShapeDtypeStruct((num_graphs, 1), jnp.float32),
        scratch_shapes=[
            pltpu.VMEM((num_graphs, h2), jnp.float32),
            pltpu.VMEM((num_graphs, 1), jnp.float32),
        ],
    )
    return final(v2, u2d, deg_t, b2.reshape(1, h2), g2.reshape(1, h2),
                 be2.reshape(1, h2), batch_pad, Wl, bl.reshape(1, 1))


# speedup vs baseline: 1.0977x; 1.0977x over previous
import dataclasses
import functools

import jax
import jax.numpy as jnp
from jax import lax
from jax.experimental import pallas as pl
from jax.experimental.pallas import tpu as pltpu
from jax.experimental.pallas import tpu_sc as plsc

HIGHEST = lax.Precision.HIGHEST

_SC_CP = pltpu.CompilerParams()
if "needs_layout_passes" in pltpu.CompilerParams.__dataclass_fields__:
    _SC_CP = dataclasses.replace(_SC_CP, needs_layout_passes=False)

NTILE = 16
NCORE = 2
CHUNK = 128
BLK = 1024


def _dot_t(a, b, precision=None):
    return lax.dot_general(a, b, (((1,), (1,)), ((), ())),
                           precision=precision,
                           preferred_element_type=jnp.float32)


def _sc_mesh():
    return plsc.VectorSubcoreMesh(core_axis_name="c", subcore_axis_name="s")


def _make_deg_kernel(n_pad, per_tile):
    nrow = n_pad // 128

    @functools.partial(
        pl.kernel,
        mesh=_sc_mesh(),
        out_type=jax.ShapeDtypeStruct((NCORE, nrow, 128), jnp.float32),
        compiler_params=_SC_CP,
        scratch_types=[
            pltpu.VMEM((per_tile,), jnp.int32),
            pltpu.VMEM((nrow, 128), jnp.float32),
            pltpu.VMEM((nrow,), jnp.int32),
            pltpu.VMEM_SHARED((nrow, 128), jnp.float32),
        ],
    )
    def deg_kernel(dst_hbm, zeros_hbm, degp_hbm, idx_v, hist_v, rows_v, acc):
        c = lax.axis_index("c")
        s = lax.axis_index("s")
        pltpu.sync_copy(dst_hbm.at[c, s], idx_v)
        pltpu.sync_copy(zeros_hbm.at[pl.ds(0, nrow)], hist_v)

        @pl.when(s == 0)
        def _():
            pltpu.sync_copy(zeros_hbm.at[pl.ds(0, nrow)], acc)

        iota = lax.iota(jnp.int32, 16)

        @pl.loop(0, nrow // 16)
        def _(j):
            rows_v[pl.ds(j * 16, 16)] = iota + j * 16

        ones = jnp.ones((16,), jnp.float32)

        @pl.loop(0, per_tile // 16)
        def _(i):
            iv = idx_v[pl.ds(i * 16, 16)]
            plsc.addupdate_scatter(
                hist_v, [lax.shift_right_logical(iv, 7), iv & 127], ones)

        plsc.subcore_barrier()
        pltpu.sync_copy(hist_v, acc.at[rows_v], add=True)
        plsc.subcore_barrier()

        @pl.when(s == 0)
        def _():
            pltpu.sync_copy(acc, degp_hbm.at[c])

    return deg_kernel


def _make_scatter_kernel(n_pad, nchunk, hh, feature_split):
    rpt = n_pad // NTILE
    sup = 40
    assert nchunk % sup == 0
    nsup = nchunk // sup

    @functools.partial(
        pl.kernel,
        mesh=_sc_mesh(),
        out_type=jax.ShapeDtypeStruct((NCORE * n_pad, hh), jnp.float32),
        scratch_types=[
            pltpu.VMEM((sup, CHUNK), jnp.int32),
            pltpu.VMEM((sup, CHUNK), jnp.int32),
            pltpu.VMEM((CHUNK, hh), jnp.float32),
            pltpu.VMEM((CHUNK, hh), jnp.float32),
            pltpu.VMEM_SHARED((n_pad, hh), jnp.float32),
            pltpu.SemaphoreType.DMA,
            pltpu.SemaphoreType.DMA,
        ],
    )
    def scatter_kernel(u_hbm, src_hbm, dst_hbm, zeros_hbm, v_hbm,
                       src_v, dst_v, g0, g1, acc, gsem0, gsem1):
        c = lax.axis_index("c")
        s = lax.axis_index("s")
        if feature_split:
            pltpu.sync_copy(u_hbm.at[pl.ds(c * n_pad + s * rpt, rpt)],
                            acc.at[pl.ds(s * rpt, rpt)])
        else:
            pltpu.sync_copy(zeros_hbm, acc.at[pl.ds(s * rpt, rpt)])
        plsc.subcore_barrier()

        def wait_gather(buf, sem):
            pltpu.make_async_copy(u_hbm.at[src_v.at[0]], buf, sem).wait()

        @pl.loop(0, nsup)
        def _(sc):
            pltpu.sync_copy(src_hbm.at[c, s, pl.ds(sc * sup, sup)], src_v)
            pltpu.sync_copy(dst_hbm.at[c, s, pl.ds(sc * sup, sup)], dst_v)
            pltpu.async_copy(u_hbm.at[src_v.at[0]], g0, gsem0)

            @pl.loop(0, sup, step=2)
            def _(j):
                pltpu.async_copy(u_hbm.at[src_v.at[j + 1]], g1, gsem1)
                wait_gather(g0, gsem0)
                pltpu.sync_copy(g0, acc.at[dst_v.at[j]], add=True)

                @pl.when(j + 2 < sup)
                def _():
                    pltpu.async_copy(u_hbm.at[src_v.at[j + 2]], g0, gsem0)

                wait_gather(g1, gsem1)
                pltpu.sync_copy(g1, acc.at[dst_v.at[j + 1]], add=True)

        plsc.subcore_barrier()
        pltpu.sync_copy(acc.at[pl.ds(s * rpt, rpt)],
                        v_hbm.at[pl.ds(c * n_pad + s * rpt, rpt)])

    return scatter_kernel


def _mm1_body(x_ref, w_ref, h_ref):
    h_ref[...] = _dot_t(x_ref[...], w_ref[...])


def _dis(degp_ref):
    deg = degp_ref[:, 0:1] + degp_ref[:, 1:2] + 1.0
    return lax.rsqrt(deg)


def _scale_body(h_ref, degp_ref, u_ref, *, hh):
    u = _dis(degp_ref) * h_ref[...]
    u_ref[0] = u[:, :hh]
    u_ref[1] = u[:, hh:]


def _ln_leaky(o, g_ref, be_ref):
    mu = jnp.mean(o, axis=1, keepdims=True)
    d = o - mu
    var = jnp.mean(d * d, axis=1, keepdims=True)
    xn = d * lax.rsqrt(var + 1e-5) * g_ref[...] + be_ref[...]
    return jnp.where(xn >= 0, xn, 0.01 * xn)


def _mid_body(v_ref, degp_ref, b_ref, g_ref, be_ref, w2_ref, u_ref):
    dis = _dis(degp_ref)
    v = jnp.concatenate([v_ref[0], v_ref[1]], axis=1)
    a = _ln_leaky(dis * v + b_ref[...], g_ref, be_ref)
    u_ref[...] = dis * _dot_t(a, w2_ref[...])


def _final_body(v_ref, u_ref, degp_ref, b_ref, g_ref, be_ref, batch_ref,
                wl_ref, bl_ref, out_ref, sums_ref, cnt_ref, *, num_graphs,
                nblocks):
    i = pl.program_id(0)

    @pl.when(i == 0)
    def _():
        sums_ref[...] = jnp.zeros_like(sums_ref)
        cnt_ref[...] = jnp.zeros_like(cnt_ref)

    dis = _dis(degp_ref)
    v = v_ref[0] + v_ref[1] + u_ref[...]
    h = _ln_leaky(dis * v + b_ref[...], g_ref, be_ref)

    gid = lax.broadcasted_iota(jnp.int32, (h.shape[0], num_graphs), 1)
    onehot = (batch_ref[...] == gid).astype(jnp.float32)
    sums_ref[...] += lax.dot_general(
        onehot, h, (((0,), (0,)), ((), ())),
        precision=HIGHEST, preferred_element_type=jnp.float32)
    cnt_ref[...] += lax.dot_general(
        onehot, jnp.ones((h.shape[0], 1), jnp.float32),
        (((0,), (0,)), ((), ())),
        precision=HIGHEST, preferred_element_type=jnp.float32)

    @pl.when(i == nblocks - 1)
    def _():
        pooled = sums_ref[...] / jnp.maximum(cnt_ref[...], 1.0)
        pb = pooled.astype(jnp.bfloat16).astype(jnp.float32)
        wb = wl_ref[...].astype(jnp.bfloat16).astype(jnp.float32)
        out_ref[...] = jnp.sum(pb * wb, axis=1, keepdims=True) + bl_ref[0, 0]


def kernel(x, edge_index, batch, W1, b1, g1, be1, W2, b2, g2, be2, Wl, bl):
    n, d = x.shape
    h1 = W1.shape[0]
    h2 = W2.shape[0]
    num_graphs = 128
    e = edge_index.shape[1]

    n_pad = ((n + BLK - 1) // BLK) * BLK
    nblocks = n_pad // BLK
    rpt = n_pad // NTILE

    grain = NCORE * NTILE * CHUNK
    e_pad = ((e + grain - 1) // grain) * grain
    per_tile_m = e_pad // NTILE
    nchunk_m = per_tile_m // CHUNK
    per_tile_d = e_pad // (NCORE * NTILE)
    nchunk_d = per_tile_d // CHUNK

    src = edge_index[0]
    dst = edge_index[1]
    sink = jnp.int32(n)
    src_p = jnp.concatenate([src, jnp.zeros((e_pad - e,), jnp.int32)])
    dst_p = jnp.concatenate([dst, jnp.full((e_pad - e,), sink, jnp.int32)])

    srcs1 = jnp.stack([src_p, src_p + n_pad]).reshape(
        NCORE, NTILE, nchunk_m, CHUNK)
    dsts1 = jnp.stack([dst_p, dst_p]).reshape(NCORE, NTILE, nchunk_m, CHUNK)
    srcs2 = src_p.reshape(NCORE, NTILE, nchunk_m // 2, CHUNK)
    dsts2 = dst_p.reshape(NCORE, NTILE, nchunk_m // 2, CHUNK)
    dstd = dst_p.reshape(NCORE, NTILE, per_tile_d)

    zeros128 = jnp.zeros((rpt, 128), jnp.float32)

    x_pad = jnp.pad(x, ((0, n_pad - n), (0, 0)))
    batch_pad = jnp.pad(batch, (0, n_pad - n),
                        constant_values=-1).reshape(n_pad, 1)

    degp = _make_deg_kernel(n_pad, per_tile_d)(dstd, zeros128)
    deg_t = jnp.transpose(degp.reshape(NCORE, n_pad))

    mm1 = pl.pallas_call(
        _mm1_body,
        grid=(nblocks,),
        in_specs=[
            pl.BlockSpec((BLK, d), lambda i: (i, 0)),
            pl.BlockSpec((h1, d), lambda i: (0, 0)),
        ],
        out_specs=pl.BlockSpec((BLK, h1), lambda i: (i, 0)),
        out_shape=jax.ShapeDtypeStruct((n_pad, h1), jnp.float32),
    )
    h1_full = mm1(x_pad, W1)

    scale = pl.pallas_call(
        functools.partial(_scale_body, hh=h1 // 2),
        grid=(nblocks,),
        in_specs=[
            pl.BlockSpec((BLK, h1), lambda i: (i, 0)),
            pl.BlockSpec((BLK, NCORE), lambda i: (i, 0)),
        ],
        out_specs=pl.BlockSpec((NCORE, BLK, h1 // 2), lambda i: (0, i, 0)),
        out_shape=jax.ShapeDtypeStruct((NCORE, n_pad, h1 // 2), jnp.float32),
    )
    u1 = scale(h1_full, deg_t).reshape(NCORE * n_pad, h1 // 2)

    v1 = _make_scatter_kernel(n_pad, nchunk_m, h1 // 2, True)(
        u1, srcs1, dsts1, zeros128)
    v1 = v1.reshape(NCORE, n_pad, h1 // 2)

    mid = pl.pallas_call(
        _mid_body,
        grid=(nblocks,),
        in_specs=[
            pl.BlockSpec((NCORE, BLK, h1 // 2), lambda i: (0, i, 0)),
            pl.BlockSpec((BLK, NCORE), lambda i: (i, 0)),
            pl.BlockSpec((1, h1), lambda i: (0, 0)),
            pl.BlockSpec((1, h1), lambda i: (0, 0)),
            pl.BlockSpec((1, h1), lambda i: (0, 0)),
            pl.BlockSpec((h2, h1), lambda i: (0, 0)),
        ],
        out_specs=pl.BlockSpec((BLK, h2), lambda i: (i, 0)),
        out_shape=jax.ShapeDtypeStruct((n_pad, h2), jnp.float32),
    )
    u2 = mid(v1, deg_t, b1.reshape(1, h1), g1.reshape(1, h1),
             be1.reshape(1, h1), W2)

    v2 = _make_scatter_kernel(n_pad, nchunk_m // 2, h2, False)(
        u2, srcs2, dsts2, zeros128)
    v2 = v2.reshape(NCORE, n_pad, h2)

    final = pl.pallas_call(
        functools.partial(_final_body, num_graphs=num_graphs,
                          nblocks=nblocks),
        grid=(nblocks,),
        in_specs=[
            pl.BlockSpec((NCORE, BLK, h2), lambda i: (0, i, 0)),
            pl.BlockSpec((BLK, h2), lambda i: (i, 0)),
            pl.BlockSpec((BLK, NCORE), lambda i: (i, 0)),
            pl.BlockSpec((1, h2), lambda i: (0, 0)),
            pl.BlockSpec((1, h2), lambda i: (0, 0)),
            pl.BlockSpec((1, h2), lambda i: (0, 0)),
            pl.BlockSpec((BLK, 1), lambda i: (i, 0)),
            pl.BlockSpec((1, h2), lambda i: (0, 0)),
            pl.BlockSpec((1, 1), lambda i: (0, 0)),
        ],
        out_specs=pl.BlockSpec((num_graphs, 1), lambda i: (0, 0)),
        out_shape=jax.ShapeDtypeStruct((num_graphs, 1), jnp.float32),
        scratch_shapes=[
            pltpu.VMEM((num_graphs, h2), jnp.float32),
            pltpu.VMEM((num_graphs, 1), jnp.float32),
        ],
    )
    return final(v2, u2, deg_t, b2.reshape(1, h2), g2.reshape(1, h2),
                 be2.reshape(1, h2), batch_pad, Wl, bl.reshape(1, 1))


# speedup vs baseline: 1.2305x; 1.1210x over previous
import dataclasses
import functools

import jax
import jax.numpy as jnp
from jax import lax
from jax.experimental import pallas as pl
from jax.experimental.pallas import tpu as pltpu
from jax.experimental.pallas import tpu_sc as plsc

HIGHEST = lax.Precision.HIGHEST

_SC_CP = pltpu.CompilerParams()
if "needs_layout_passes" in pltpu.CompilerParams.__dataclass_fields__:
    _SC_CP = dataclasses.replace(_SC_CP, needs_layout_passes=False)

NTILE = 16
NCORE = 2
CHUNK = 128
BLK = 1024


def _dot_t(a, b, precision=None):
    return lax.dot_general(a, b, (((1,), (1,)), ((), ())),
                           precision=precision,
                           preferred_element_type=jnp.float32)


def _sc_mesh():
    return plsc.VectorSubcoreMesh(core_axis_name="c", subcore_axis_name="s")


def _make_deg_kernel(n_pad, per_tile):
    nrow = n_pad // 128

    @functools.partial(
        pl.kernel,
        mesh=_sc_mesh(),
        out_type=jax.ShapeDtypeStruct((NCORE, nrow, 128), jnp.float32),
        compiler_params=_SC_CP,
        scratch_types=[
            pltpu.VMEM((per_tile,), jnp.int32),
            pltpu.VMEM((nrow, 128), jnp.float32),
            pltpu.VMEM((nrow,), jnp.int32),
            pltpu.VMEM_SHARED((nrow, 128), jnp.float32),
        ],
    )
    def deg_kernel(dst_hbm, zeros_hbm, degp_hbm, idx_v, hist_v, rows_v, acc):
        c = lax.axis_index("c")
        s = lax.axis_index("s")
        pltpu.sync_copy(dst_hbm.at[c, s], idx_v)
        pltpu.sync_copy(zeros_hbm.at[pl.ds(0, nrow)], hist_v)

        @pl.when(s == 0)
        def _():
            pltpu.sync_copy(zeros_hbm.at[pl.ds(0, nrow)], acc)

        iota = lax.iota(jnp.int32, 16)

        @pl.loop(0, nrow // 16)
        def _(j):
            rows_v[pl.ds(j * 16, 16)] = iota + j * 16

        ones = jnp.ones((16,), jnp.float32)

        @pl.loop(0, per_tile // 16)
        def _(i):
            iv = idx_v[pl.ds(i * 16, 16)]
            plsc.addupdate_scatter(
                hist_v, [lax.shift_right_logical(iv, 7), iv & 127], ones)

        plsc.subcore_barrier()
        pltpu.sync_copy(hist_v, acc.at[rows_v], add=True)
        plsc.subcore_barrier()

        @pl.when(s == 0)
        def _():
            pltpu.sync_copy(acc, degp_hbm.at[c])

    return deg_kernel


def _make_scatter_kernel(n_pad, nchunk, hh, feature_split):
    rpt = n_pad // NTILE
    sup = 40
    assert nchunk % sup == 0
    nsup = nchunk // sup

    @functools.partial(
        pl.kernel,
        mesh=_sc_mesh(),
        out_type=jax.ShapeDtypeStruct((NCORE * n_pad, hh), jnp.float32),
        scratch_types=[
            pltpu.VMEM((sup, CHUNK), jnp.int32),
            pltpu.VMEM((sup, CHUNK), jnp.int32),
            pltpu.VMEM((CHUNK, hh), jnp.float32),
            pltpu.VMEM((CHUNK, hh), jnp.float32),
            pltpu.VMEM_SHARED((n_pad, hh), jnp.float32),
            pltpu.SemaphoreType.DMA,
            pltpu.SemaphoreType.DMA,
        ],
    )
    def scatter_kernel(u_hbm, src_hbm, dst_hbm, zeros_hbm, v_hbm,
                       src_v, dst_v, g0, g1, acc, gsem0, gsem1):
        c = lax.axis_index("c")
        s = lax.axis_index("s")
        if feature_split:
            pltpu.sync_copy(u_hbm.at[pl.ds(c * n_pad + s * rpt, rpt)],
                            acc.at[pl.ds(s * rpt, rpt)])
        else:
            pltpu.sync_copy(zeros_hbm, acc.at[pl.ds(s * rpt, rpt)])
        plsc.subcore_barrier()

        def wait_gather(buf, sem):
            pltpu.make_async_copy(u_hbm.at[src_v.at[0]], buf, sem).wait()

        @pl.loop(0, nsup)
        def _(sc):
            pltpu.sync_copy(src_hbm.at[c, s, pl.ds(sc * sup, sup)], src_v)
            pltpu.sync_copy(dst_hbm.at[c, s, pl.ds(sc * sup, sup)], dst_v)
            pltpu.async_copy(u_hbm.at[src_v.at[0]], g0, gsem0)

            @pl.loop(0, sup, step=2)
            def _(j):
                pltpu.async_copy(u_hbm.at[src_v.at[j + 1]], g1, gsem1)
                wait_gather(g0, gsem0)
                pltpu.sync_copy(g0, acc.at[dst_v.at[j]], add=True)

                @pl.when(j + 2 < sup)
                def _():
                    pltpu.async_copy(u_hbm.at[src_v.at[j + 2]], g0, gsem0)

                wait_gather(g1, gsem1)
                pltpu.sync_copy(g1, acc.at[dst_v.at[j + 1]], add=True)

        plsc.subcore_barrier()
        pltpu.sync_copy(acc.at[pl.ds(s * rpt, rpt)],
                        v_hbm.at[pl.ds(c * n_pad + s * rpt, rpt)])

    return scatter_kernel


def _first_body(x_ref, w_ref, degp_ref, u_ref, *, hh):
    u = _dis(degp_ref) * _dot_t(x_ref[...], w_ref[...])
    u_ref[0] = u[:, :hh]
    u_ref[1] = u[:, hh:]


def _dis(degp_ref):
    deg = degp_ref[:, 0:1] + degp_ref[:, 1:2] + 1.0
    return lax.rsqrt(deg)


def _scale_body(h_ref, degp_ref, u_ref, *, hh):
    u = _dis(degp_ref) * h_ref[...]
    u_ref[0] = u[:, :hh]
    u_ref[1] = u[:, hh:]


def _ln_leaky(o, g_ref, be_ref):
    mu = jnp.mean(o, axis=1, keepdims=True)
    d = o - mu
    var = jnp.mean(d * d, axis=1, keepdims=True)
    xn = d * lax.rsqrt(var + 1e-5) * g_ref[...] + be_ref[...]
    return jnp.where(xn >= 0, xn, 0.01 * xn)


def _mid_body(v_ref, degp_ref, b_ref, g_ref, be_ref, w2_ref, u_ref):
    dis = _dis(degp_ref)
    v = jnp.concatenate([v_ref[0], v_ref[1]], axis=1)
    a = _ln_leaky(dis * v + b_ref[...], g_ref, be_ref)
    u_ref[...] = dis * _dot_t(a, w2_ref[...])


def _final_body(v_ref, u_ref, degp_ref, b_ref, g_ref, be_ref, batch_ref,
                wl_ref, bl_ref, out_ref, sums_ref, cnt_ref, *, num_graphs,
                nblocks):
    i = pl.program_id(0)

    @pl.when(i == 0)
    def _():
        sums_ref[...] = jnp.zeros_like(sums_ref)
        cnt_ref[...] = jnp.zeros_like(cnt_ref)

    dis = _dis(degp_ref)
    v = v_ref[0] + v_ref[1] + u_ref[...]
    h = _ln_leaky(dis * v + b_ref[...], g_ref, be_ref)

    gid = lax.broadcasted_iota(jnp.int32, (h.shape[0], num_graphs), 1)
    onehot = (batch_ref[...] == gid).astype(jnp.float32)
    sums_ref[...] += lax.dot_general(
        onehot, h, (((0,), (0,)), ((), ())),
        precision=HIGHEST, preferred_element_type=jnp.float32)
    cnt_ref[...] += lax.dot_general(
        onehot, jnp.ones((h.shape[0], 1), jnp.float32),
        (((0,), (0,)), ((), ())),
        precision=HIGHEST, preferred_element_type=jnp.float32)

    @pl.when(i == nblocks - 1)
    def _():
        pooled = sums_ref[...] / jnp.maximum(cnt_ref[...], 1.0)
        pb = pooled.astype(jnp.bfloat16).astype(jnp.float32)
        wb = wl_ref[...].astype(jnp.bfloat16).astype(jnp.float32)
        out_ref[...] = jnp.sum(pb * wb, axis=1, keepdims=True) + bl_ref[0, 0]


def kernel(x, edge_index, batch, W1, b1, g1, be1, W2, b2, g2, be2, Wl, bl):
    n, d = x.shape
    h1 = W1.shape[0]
    h2 = W2.shape[0]
    num_graphs = 128
    e = edge_index.shape[1]

    n_pad = ((n + BLK - 1) // BLK) * BLK
    nblocks = n_pad // BLK
    rpt = n_pad // NTILE

    grain = NCORE * NTILE * CHUNK
    e_pad = ((e + grain - 1) // grain) * grain
    per_tile_m = e_pad // NTILE
    nchunk_m = per_tile_m // CHUNK
    per_tile_d = e_pad // (NCORE * NTILE)
    nchunk_d = per_tile_d // CHUNK

    src = edge_index[0]
    dst = edge_index[1]
    sink = jnp.int32(n)
    src_p = jnp.concatenate([src, jnp.zeros((e_pad - e,), jnp.int32)])
    dst_p = jnp.concatenate([dst, jnp.full((e_pad - e,), sink, jnp.int32)])

    srcs1 = jnp.stack([src_p, src_p + n_pad]).reshape(
        NCORE, NTILE, nchunk_m, CHUNK)
    dsts1 = jnp.stack([dst_p, dst_p]).reshape(NCORE, NTILE, nchunk_m, CHUNK)
    srcs2 = src_p.reshape(NCORE, NTILE, nchunk_m // 2, CHUNK)
    dsts2 = dst_p.reshape(NCORE, NTILE, nchunk_m // 2, CHUNK)
    dstd = dst_p.reshape(NCORE, NTILE, per_tile_d)

    zeros128 = jnp.zeros((rpt, 128), jnp.float32)

    x_pad = jnp.pad(x, ((0, n_pad - n), (0, 0)))
    batch_pad = jnp.pad(batch, (0, n_pad - n),
                        constant_values=-1).reshape(n_pad, 1)

    degp = _make_deg_kernel(n_pad, per_tile_d)(dstd, zeros128)
    deg_t = jnp.transpose(degp.reshape(NCORE, n_pad))

    first = pl.pallas_call(
        functools.partial(_first_body, hh=h1 // 2),
        grid=(nblocks,),
        in_specs=[
            pl.BlockSpec((BLK, d), lambda i: (i, 0)),
            pl.BlockSpec((h1, d), lambda i: (0, 0)),
            pl.BlockSpec((BLK, NCORE), lambda i: (i, 0)),
        ],
        out_specs=pl.BlockSpec((NCORE, BLK, h1 // 2), lambda i: (0, i, 0)),
        out_shape=jax.ShapeDtypeStruct((NCORE, n_pad, h1 // 2), jnp.float32),
    )
    u1 = first(x_pad, W1, deg_t).reshape(NCORE * n_pad, h1 // 2)

    v1 = _make_scatter_kernel(n_pad, nchunk_m, h1 // 2, True)(
        u1, srcs1, dsts1, zeros128)
    v1 = v1.reshape(NCORE, n_pad, h1 // 2)

    mid = pl.pallas_call(
        _mid_body,
        grid=(nblocks,),
        in_specs=[
            pl.BlockSpec((NCORE, BLK, h1 // 2), lambda i: (0, i, 0)),
            pl.BlockSpec((BLK, NCORE), lambda i: (i, 0)),
            pl.BlockSpec((1, h1), lambda i: (0, 0)),
            pl.BlockSpec((1, h1), lambda i: (0, 0)),
            pl.BlockSpec((1, h1), lambda i: (0, 0)),
            pl.BlockSpec((h2, h1), lambda i: (0, 0)),
        ],
        out_specs=pl.BlockSpec((BLK, h2), lambda i: (i, 0)),
        out_shape=jax.ShapeDtypeStruct((n_pad, h2), jnp.float32),
    )
    u2 = mid(v1, deg_t, b1.reshape(1, h1), g1.reshape(1, h1),
             be1.reshape(1, h1), W2)

    v2 = _make_scatter_kernel(n_pad, nchunk_m // 2, h2, False)(
        u2, srcs2, dsts2, zeros128)
    v2 = v2.reshape(NCORE, n_pad, h2)

    final = pl.pallas_call(
        functools.partial(_final_body, num_graphs=num_graphs,
                          nblocks=nblocks),
        grid=(nblocks,),
        in_specs=[
            pl.BlockSpec((NCORE, BLK, h2), lambda i: (0, i, 0)),
            pl.BlockSpec((BLK, h2), lambda i: (i, 0)),
            pl.BlockSpec((BLK, NCORE), lambda i: (i, 0)),
            pl.BlockSpec((1, h2), lambda i: (0, 0)),
            pl.BlockSpec((1, h2), lambda i: (0, 0)),
            pl.BlockSpec((1, h2), lambda i: (0, 0)),
            pl.BlockSpec((BLK, 1), lambda i: (i, 0)),
            pl.BlockSpec((1, h2), lambda i: (0, 0)),
            pl.BlockSpec((1, 1), lambda i: (0, 0)),
        ],
        out_specs=pl.BlockSpec((num_graphs, 1), lambda i: (0, 0)),
        out_shape=jax.ShapeDtypeStruct((num_graphs, 1), jnp.float32),
        scratch_shapes=[
            pltpu.VMEM((num_graphs, h2), jnp.float32),
            pltpu.VMEM((num_graphs, 1), jnp.float32),
        ],
    )
    return final(v2, u2, deg_t, b2.reshape(1, h2), g2.reshape(1, h2),
                 be2.reshape(1, h2), batch_pad, Wl, bl.reshape(1, 1))
